# Initial kernel scaffold; baseline (speedup 1.0000x reference)
import jax
import jax.numpy as jnp
from jax.experimental import pallas as pl


def kernel(x, edge_index, W1, b1, W2, b2, Wl, bl):
    raise NotImplementedError("write your pallas kernel here")


# speedup vs baseline: 28.0467x; 28.0467x over previous
import functools

import jax
import jax.numpy as jnp
from jax import lax
from jax.experimental import pallas as pl
from jax.experimental.pallas import tpu as pltpu
from jax.experimental.pallas import tpu_sc as plsc

NC = 2
NS = 16
NW = NC * NS
CH = 128


def _mesh():
    return plsc.VectorSubcoreMesh(
        core_axis_name="c", subcore_axis_name="s", num_cores=NC,
        num_subcores=NS)


_SC_PARAMS = pltpu.CompilerParams(use_tc_tiling_on_sc=False)


def _zero_vmem(ref, rows, cols):
    z = jnp.zeros((16,), jnp.float32)

    def row(i, _):
        for k in range(cols // 16):
            ref[i, pl.ds(k * 16, 16)] = z
        return _

    lax.fori_loop(0, rows, row, 0)


def _make_deg(n_chunks, npad):
    cmax = -(-n_chunks // NW)
    per_tile = npad // NS

    @functools.partial(
        pl.kernel,
        out_type=jax.ShapeDtypeStruct((NC, npad), jnp.float32),
        mesh=_mesh(),
        scratch_types=[
            pltpu.VMEM((cmax, CH), jnp.int32),
            pltpu.VMEM((CH,), jnp.float32),
            pltpu.VMEM((per_tile,), jnp.float32),
            pltpu.VMEM_SHARED((npad,), jnp.float32),
        ],
        compiler_params=_SC_PARAMS,
    )
    def deg_kernel(dst_hbm, out_hbm, idx_v, ones_v, stage_v, deg_sh):
        c = lax.axis_index("c")
        s = lax.axis_index("s")
        wid = c * NS + s
        start = wid * n_chunks // NW
        count = (wid + 1) * n_chunks // NW - start

        one = jnp.ones((16,), jnp.float32)
        z = jnp.zeros((16,), jnp.float32)
        for k in range(CH // 16):
            ones_v[pl.ds(k * 16, 16)] = one

        def zrow(i, carry):
            stage_v[pl.ds(i * 16, 16)] = z
            return carry

        lax.fori_loop(0, per_tile // 16, zrow, 0)
        pltpu.sync_copy(stage_v, deg_sh.at[pl.ds(s * per_tile, per_tile)])
        plsc.subcore_barrier()

        pltpu.sync_copy(dst_hbm.at[pl.ds(start, cmax)], idx_v)

        def body(j, carry):
            @pl.when(j < count)
            def _():
                pltpu.sync_copy(ones_v, deg_sh.at[idx_v.at[j]], add=True)
            return carry

        lax.fori_loop(0, cmax, body, 0)
        plsc.subcore_barrier()
        pltpu.sync_copy(deg_sh.at[pl.ds(s * per_tile, per_tile)], stage_v)
        pltpu.sync_copy(stage_v, out_hbm.at[c, pl.ds(s * per_tile, per_tile)])

    return deg_kernel


GROUP = 8


def _make_prop(n, d, n_chunks):
    rows_per_tile = n // NS
    zr = rows_per_tile // 5
    assert rows_per_tile % 5 == 0 and zr <= CH

    @functools.partial(
        pl.kernel,
        out_type=jax.ShapeDtypeStruct((NC, n, d), jnp.float32),
        mesh=_mesh(),
        scratch_types=[
            pltpu.VMEM((2, GROUP, CH), jnp.int32),
            pltpu.VMEM((2, GROUP, CH), jnp.int32),
            pltpu.VMEM((CH, d), jnp.float32),
            pltpu.VMEM((CH, d), jnp.float32),
            pltpu.VMEM_SHARED((n, d), jnp.float32),
            pltpu.SemaphoreType.DMA,
            pltpu.SemaphoreType.DMA,
        ],
        compiler_params=_SC_PARAMS,
    )
    def prop_kernel(g_hbm, src_hbm, dst_hbm, out_hbm, src_v, dst_v,
                    rows_a, rows_b, acc_sh, sem_a, sem_b):
        c = lax.axis_index("c")
        s = lax.axis_index("s")
        wid = c * NS + s
        start = wid * n_chunks // NW
        count = (wid + 1) * n_chunks // NW - start

        _zero_vmem(rows_a, CH, d)
        for k in range(5):
            pltpu.sync_copy(
                rows_a.at[pl.ds(0, zr)],
                acc_sh.at[pl.ds(s * rows_per_tile + k * zr, zr)])
        plsc.subcore_barrier()

        pltpu.sync_copy(src_hbm.at[pl.ds(start, GROUP)], src_v.at[0])
        pltpu.sync_copy(dst_hbm.at[pl.ds(start, GROUP)], dst_v.at[0])
        pltpu.async_copy(g_hbm.at[src_v.at[0, 0]], rows_a, sem_a)

        def body(j, carry):
            p = lax.rem(j // GROUP, 2)
            jj = lax.rem(j, GROUP)

            @pl.when(jj == 0)
            def _():
                pltpu.sync_copy(
                    src_hbm.at[pl.ds(start + j + GROUP, GROUP)],
                    src_v.at[1 - p])
                pltpu.sync_copy(
                    dst_hbm.at[pl.ds(start + j + GROUP, GROUP)],
                    dst_v.at[1 - p])

            @pl.when(j + 1 < count)
            def _():
                pn = lax.rem((j + 1) // GROUP, 2)
                jn = lax.rem(j + 1, GROUP)

                @pl.when(lax.rem(j, 2) == 0)
                def _():
                    pltpu.async_copy(g_hbm.at[src_v.at[pn, jn]], rows_b,
                                     sem_b)

                @pl.when(lax.rem(j, 2) == 1)
                def _():
                    pltpu.async_copy(g_hbm.at[src_v.at[pn, jn]], rows_a,
                                     sem_a)

            @pl.when(lax.rem(j, 2) == 0)
            def _():
                pltpu.make_async_copy(g_hbm.at[src_v.at[p, jj]], rows_a,
                                      sem_a).wait()
                pltpu.sync_copy(rows_a, acc_sh.at[dst_v.at[p, jj]], add=True)

            @pl.when(lax.rem(j, 2) == 1)
            def _():
                pltpu.make_async_copy(g_hbm.at[src_v.at[p, jj]], rows_b,
                                      sem_b).wait()
                pltpu.sync_copy(rows_b, acc_sh.at[dst_v.at[p, jj]], add=True)

            return carry

        lax.fori_loop(0, count, body, 0)
        plsc.subcore_barrier()

        for k in range(5):
            off = s * rows_per_tile + k * zr
            pltpu.sync_copy(acc_sh.at[pl.ds(off, zr)], rows_a.at[pl.ds(0, zr)])
            pltpu.sync_copy(rows_a.at[pl.ds(0, zr)], out_hbm.at[c, pl.ds(off, zr)])

    return prop_kernel


def _prep_call(deg2, x, n, npad, d):
    def body(deg_ref, x_ref, dinv_ref, g_ref):
        total = deg_ref[0] + deg_ref[1] + 1.0
        dinv = lax.rsqrt(total)
        dinv_ref[...] = dinv
        g_ref[...] = x_ref[...] * dinv[:n]

    return pl.pallas_call(
        body,
        out_shape=[
            jax.ShapeDtypeStruct((npad, 1), jnp.float32),
            jax.ShapeDtypeStruct((n, d), jnp.float32),
        ],
    )(deg2, x)


def _layer_call(s2, g, dinv, w, b, relu, scale_out, n, d):
    rb = 2000

    def body(s_ref, g_ref, dinv_ref, w_ref, b_ref, o_ref):
        u = (s_ref[0] + s_ref[1] + g_ref[...]) * dinv_ref[...]
        t = jnp.dot(u, w_ref[...], preferred_element_type=jnp.float32)
        t = t + b_ref[...]
        if relu:
            t = jnp.maximum(t, 0.0)
        if scale_out:
            t = t * dinv_ref[...]
        o_ref[...] = t

    return pl.pallas_call(
        body,
        grid=(n // rb,),
        in_specs=[
            pl.BlockSpec((2, rb, d), lambda i: (0, i, 0)),
            pl.BlockSpec((rb, d), lambda i: (i, 0)),
            pl.BlockSpec((rb, 1), lambda i: (i, 0)),
            pl.BlockSpec((d, d), lambda i: (0, 0)),
            pl.BlockSpec((1, d), lambda i: (0, 0)),
        ],
        out_specs=pl.BlockSpec((rb, d), lambda i: (i, 0)),
        out_shape=jax.ShapeDtypeStruct((n, d), jnp.float32),
    )(s2, g, dinv, w, b)


def kernel(x, edge_index, W1, b1, W2, b2, Wl, bl):
    n, d = x.shape
    e = edge_index.shape[1]
    assert e % CH == 0
    n_chunks = e // CH
    npad = -(-n // (NS * 16)) * (NS * 16)

    starts = [w * n_chunks // NW for w in range(NW)]
    counts = [(w + 1) * n_chunks // NW - starts[w] for w in range(NW)]
    npc = max(
        max(starts[w] + ((counts[w] - 1) // GROUP) * GROUP + 2 * GROUP
            for w in range(NW)),
        n_chunks)

    src = edge_index[0].astype(jnp.int32).reshape(n_chunks, CH)
    dst = edge_index[1].astype(jnp.int32).reshape(n_chunks, CH)
    if npc > n_chunks:
        pad = ((0, npc - n_chunks), (0, 0))
        src = jnp.pad(src, pad)
        dst = jnp.pad(dst, pad)

    deg2 = _make_deg(n_chunks, npad)(dst)
    dinv_p, g = _prep_call(deg2.reshape(NC, npad, 1), x, n, npad, d)
    dinv = dinv_p[:n]

    prop = _make_prop(n, d, n_chunks)
    b1r = b1.reshape(1, d)
    b2r = b2.reshape(1, d)
    blr = bl.reshape(1, d)

    s2 = prop(g, src, dst)
    g = _layer_call(s2, g, dinv, W1, b1r, True, True, n, d)
    s2 = prop(g, src, dst)
    g = _layer_call(s2, g, dinv, W2, b2r, True, True, n, d)
    s2 = prop(g, src, dst)
    out = _layer_call(s2, g, dinv, Wl, blr, False, False, n, d)
    return out


# speedup vs baseline: 29.5973x; 1.0553x over previous
import functools

import jax
import jax.numpy as jnp
from jax import lax
from jax.experimental import pallas as pl
from jax.experimental.pallas import tpu as pltpu
from jax.experimental.pallas import tpu_sc as plsc

NC = 2
NS = 16
NW = NC * NS
CH = 128


def _mesh():
    return plsc.VectorSubcoreMesh(
        core_axis_name="c", subcore_axis_name="s", num_cores=NC,
        num_subcores=NS)


_SC_PARAMS = pltpu.CompilerParams(use_tc_tiling_on_sc=False)


def _zero_vmem(ref, rows, cols):
    z = jnp.zeros((16,), jnp.float32)

    def row(i, _):
        for k in range(cols // 16):
            ref[i, pl.ds(k * 16, 16)] = z
        return _

    lax.fori_loop(0, rows, row, 0)


def _make_deg(n_chunks, npad):
    cmax = -(-n_chunks // NW)
    per_tile = npad // NS

    @functools.partial(
        pl.kernel,
        out_type=jax.ShapeDtypeStruct((NC, npad), jnp.float32),
        mesh=_mesh(),
        scratch_types=[
            pltpu.VMEM((cmax, CH), jnp.int32),
            pltpu.VMEM((CH,), jnp.float32),
            pltpu.VMEM((per_tile,), jnp.float32),
            pltpu.VMEM_SHARED((npad,), jnp.float32),
        ],
        compiler_params=_SC_PARAMS,
    )
    def deg_kernel(dst_hbm, out_hbm, idx_v, ones_v, stage_v, deg_sh):
        c = lax.axis_index("c")
        s = lax.axis_index("s")
        wid = c * NS + s
        start = wid * n_chunks // NW
        count = (wid + 1) * n_chunks // NW - start

        one = jnp.ones((16,), jnp.float32)
        z = jnp.zeros((16,), jnp.float32)
        for k in range(CH // 16):
            ones_v[pl.ds(k * 16, 16)] = one

        def zrow(i, carry):
            stage_v[pl.ds(i * 16, 16)] = z
            return carry

        lax.fori_loop(0, per_tile // 16, zrow, 0)
        pltpu.sync_copy(stage_v, deg_sh.at[pl.ds(s * per_tile, per_tile)])
        plsc.subcore_barrier()

        pltpu.sync_copy(dst_hbm.at[pl.ds(start, cmax)], idx_v)

        def body(j, carry):
            @pl.when(j < count)
            def _():
                pltpu.sync_copy(ones_v, deg_sh.at[idx_v.at[j]], add=True)
            return carry

        lax.fori_loop(0, cmax, body, 0)
        plsc.subcore_barrier()
        pltpu.sync_copy(deg_sh.at[pl.ds(s * per_tile, per_tile)], stage_v)
        pltpu.sync_copy(stage_v, out_hbm.at[c, pl.ds(s * per_tile, per_tile)])

    return deg_kernel


GROUP = 16


def _make_prop(n, d, n_chunks):
    rows_per_tile = n // NS
    zr = rows_per_tile // 5
    assert rows_per_tile % 5 == 0 and zr <= CH

    @functools.partial(
        pl.kernel,
        out_type=jax.ShapeDtypeStruct((NC, n, d), jnp.float32),
        mesh=_mesh(),
        scratch_types=[
            pltpu.VMEM((2, GROUP, CH), jnp.int32),
            pltpu.VMEM((2, GROUP, CH), jnp.int32),
            pltpu.VMEM((CH, d), jnp.float32),
            pltpu.VMEM((CH, d), jnp.float32),
            pltpu.VMEM_SHARED((n, d), jnp.float32),
            pltpu.SemaphoreType.DMA,
            pltpu.SemaphoreType.DMA,
            pltpu.SemaphoreType.DMA,
            pltpu.SemaphoreType.DMA,
        ],
        compiler_params=_SC_PARAMS,
    )
    def prop_kernel(g_hbm, src_hbm, dst_hbm, out_hbm, src_v, dst_v,
                    rows_a, rows_b, acc_sh, sem_a, sem_b, sem_sa, sem_sb):
        c = lax.axis_index("c")
        s = lax.axis_index("s")
        wid = c * NS + s
        start = wid * n_chunks // NW
        count = (wid + 1) * n_chunks // NW - start

        _zero_vmem(rows_a, CH, d)
        for k in range(5):
            pltpu.sync_copy(
                rows_a.at[pl.ds(0, zr)],
                acc_sh.at[pl.ds(s * rows_per_tile + k * zr, zr)])
        plsc.subcore_barrier()

        pltpu.sync_copy(src_hbm.at[pl.ds(start, GROUP)], src_v.at[0])
        pltpu.sync_copy(dst_hbm.at[pl.ds(start, GROUP)], dst_v.at[0])
        pltpu.async_copy(g_hbm.at[src_v.at[0, 0]], rows_a, sem_a)

        def body(j, carry):
            p = lax.rem(j // GROUP, 2)
            jj = lax.rem(j, GROUP)

            @pl.when(jj == 0)
            def _():
                pltpu.sync_copy(
                    src_hbm.at[pl.ds(start + j + GROUP, GROUP)],
                    src_v.at[1 - p])
                pltpu.sync_copy(
                    dst_hbm.at[pl.ds(start + j + GROUP, GROUP)],
                    dst_v.at[1 - p])

            @pl.when(j + 1 < count)
            def _():
                pn = lax.rem((j + 1) // GROUP, 2)
                jn = lax.rem(j + 1, GROUP)

                @pl.when(lax.rem(j, 2) == 0)
                def _():
                    @pl.when(j > 0)
                    def _():
                        pltpu.make_async_copy(
                            rows_b, acc_sh.at[dst_v.at[0, 0]], sem_sb).wait()
                    pltpu.async_copy(g_hbm.at[src_v.at[pn, jn]], rows_b,
                                     sem_b)

                @pl.when(lax.rem(j, 2) == 1)
                def _():
                    pltpu.make_async_copy(
                        rows_a, acc_sh.at[dst_v.at[0, 0]], sem_sa).wait()
                    pltpu.async_copy(g_hbm.at[src_v.at[pn, jn]], rows_a,
                                     sem_a)

            @pl.when(lax.rem(j, 2) == 0)
            def _():
                pltpu.make_async_copy(g_hbm.at[src_v.at[p, jj]], rows_a,
                                      sem_a).wait()
                pltpu.make_async_copy(
                    rows_a, acc_sh.at[dst_v.at[p, jj]], sem_sa).start(add=True)

            @pl.when(lax.rem(j, 2) == 1)
            def _():
                pltpu.make_async_copy(g_hbm.at[src_v.at[p, jj]], rows_b,
                                      sem_b).wait()
                pltpu.make_async_copy(
                    rows_b, acc_sh.at[dst_v.at[p, jj]], sem_sb).start(add=True)

            return carry

        lax.fori_loop(0, count, body, 0)
        pltpu.make_async_copy(rows_a, acc_sh.at[dst_v.at[0, 0]], sem_sa).wait()
        pltpu.make_async_copy(rows_b, acc_sh.at[dst_v.at[0, 0]], sem_sb).wait()
        plsc.subcore_barrier()

        for k in range(5):
            off = s * rows_per_tile + k * zr
            pltpu.sync_copy(acc_sh.at[pl.ds(off, zr)],
                            out_hbm.at[c, pl.ds(off, zr)])

    return prop_kernel


def _prep_call(deg2, x, n, npad, d):
    def body(deg_ref, x_ref, dinv_ref, g_ref):
        total = deg_ref[0] + deg_ref[1] + 1.0
        dinv = lax.rsqrt(total)
        dinv_ref[...] = dinv
        g_ref[...] = x_ref[...] * dinv[:n]

    return pl.pallas_call(
        body,
        out_shape=[
            jax.ShapeDtypeStruct((npad, 1), jnp.float32),
            jax.ShapeDtypeStruct((n, d), jnp.float32),
        ],
    )(deg2, x)


def _layer_call(s2, g, dinv, w, b, relu, scale_out, n, d):
    rb = 2000

    def body(s_ref, g_ref, dinv_ref, w_ref, b_ref, o_ref):
        u = (s_ref[0] + s_ref[1] + g_ref[...]) * dinv_ref[...]
        t = jnp.dot(u, w_ref[...], preferred_element_type=jnp.float32)
        t = t + b_ref[...]
        if relu:
            t = jnp.maximum(t, 0.0)
        if scale_out:
            t = t * dinv_ref[...]
        o_ref[...] = t

    return pl.pallas_call(
        body,
        grid=(n // rb,),
        in_specs=[
            pl.BlockSpec((2, rb, d), lambda i: (0, i, 0)),
            pl.BlockSpec((rb, d), lambda i: (i, 0)),
            pl.BlockSpec((rb, 1), lambda i: (i, 0)),
            pl.BlockSpec((d, d), lambda i: (0, 0)),
            pl.BlockSpec((1, d), lambda i: (0, 0)),
        ],
        out_specs=pl.BlockSpec((rb, d), lambda i: (i, 0)),
        out_shape=jax.ShapeDtypeStruct((n, d), jnp.float32),
    )(s2, g, dinv, w, b)


def kernel(x, edge_index, W1, b1, W2, b2, Wl, bl):
    n, d = x.shape
    e = edge_index.shape[1]
    assert e % CH == 0
    n_chunks = e // CH
    npad = -(-n // (NS * 16)) * (NS * 16)

    starts = [w * n_chunks // NW for w in range(NW)]
    counts = [(w + 1) * n_chunks // NW - starts[w] for w in range(NW)]
    npc = max(
        max(starts[w] + ((counts[w] - 1) // GROUP) * GROUP + 2 * GROUP
            for w in range(NW)),
        n_chunks)

    src = edge_index[0].astype(jnp.int32).reshape(n_chunks, CH)
    dst = edge_index[1].astype(jnp.int32).reshape(n_chunks, CH)
    if npc > n_chunks:
        pad = ((0, npc - n_chunks), (0, 0))
        src = jnp.pad(src, pad)
        dst = jnp.pad(dst, pad)

    deg2 = _make_deg(n_chunks, npad)(dst)
    dinv_p, g = _prep_call(deg2.reshape(NC, npad, 1), x, n, npad, d)
    dinv = dinv_p[:n]

    prop = _make_prop(n, d, n_chunks)
    b1r = b1.reshape(1, d)
    b2r = b2.reshape(1, d)
    blr = bl.reshape(1, d)

    s2 = prop(g, src, dst)
    g = _layer_call(s2, g, dinv, W1, b1r, True, True, n, d)
    s2 = prop(g, src, dst)
    g = _layer_call(s2, g, dinv, W2, b2r, True, True, n, d)
    s2 = prop(g, src, dst)
    out = _layer_call(s2, g, dinv, Wl, blr, False, False, n, d)
    return out
